# Initial kernel scaffold; baseline (speedup 1.0000x reference)
import jax
import jax.numpy as jnp
from jax.experimental import pallas as pl


def kernel(edge_index, h, W_emb, Wq, bq, Wk, bk, Wv, bv, Wo, bo, ln1_g, ln1_b, W1, b1, W2, b2, ln2_g, ln2_b):
    raise NotImplementedError("write your pallas kernel here")


# speedup vs baseline: 44.8827x; 44.8827x over previous
import functools

import jax
import jax.numpy as jnp
from jax import lax
from jax.experimental import pallas as pl
from jax.experimental.pallas import tpu as pltpu
from jax.experimental.pallas import tpu_sc as plsc

N = 10000
E = 320000
D = 128
H = 8
DH = 16
ZPAD = 16
ROWW = D + ZPAD

NC = 2
NS = 16
NW = NC * NS
EPW = E // NW
CHUNK = 40
NCH = EPW // CHUNK
ZR = 40
NRB = N // ZR
RBIT = -(-NRB // NS)
BN = 1000
GRID = N // BN


def _edge_body(src_hbm, dst_hbm, q_hbm, k_hbm, v_hbm, out_hbm, acc):
    pl.run_scoped(
        functools.partial(_edge_inner, src_hbm, dst_hbm, q_hbm, k_hbm,
                          v_hbm, out_hbm, acc),
        pltpu.VMEM((CHUNK,), jnp.int32),
        pltpu.VMEM((CHUNK,), jnp.int32),
        pltpu.VMEM((CHUNK, D), jnp.float32),
        pltpu.VMEM((CHUNK, D), jnp.float32),
        pltpu.VMEM((CHUNK, D), jnp.float32),
        pltpu.VMEM((CHUNK, ROWW), jnp.float32),
        pltpu.VMEM((ZR, ROWW), jnp.float32),
        pltpu.SemaphoreType.DMA,
        pltpu.SemaphoreType.DMA,
        pltpu.SemaphoreType.DMA,
    )


def _edge_inner(src_hbm, dst_hbm, q_hbm, k_hbm, v_hbm, out_hbm, acc,
                sidx, didx, kbuf, qbuf, vbuf, obuf, zbuf,
                sem1, sem2, sem3):
    c = lax.axis_index("c")
    s = lax.axis_index("s")
    w = c * NS + s
    ebase = w * EPW
    lanes = lax.broadcasted_iota(jnp.int32, (16,), 0)

    def zrow(i, carry):
        for j in range(ROWW // 16):
            zbuf[i, pl.ds(j * 16, 16)] = jnp.zeros((16,), jnp.float32)
        return carry

    lax.fori_loop(0, ZR, zrow, 0)
    for it in range(RBIT):
        b = it * NS + s
        @pl.when(b < NRB)
        def _():
            pltpu.sync_copy(zbuf, acc.at[pl.ds(b * ZR, ZR)])
    plsc.subcore_barrier()

    def chunk_body(kc, carry):
        base = ebase + kc * CHUNK
        pltpu.sync_copy(src_hbm.at[pl.ds(base, CHUNK)], sidx)
        pltpu.sync_copy(dst_hbm.at[pl.ds(base, CHUNK)], didx)
        cp1 = pltpu.async_copy(k_hbm.at[sidx], kbuf, sem1)
        cp2 = pltpu.async_copy(q_hbm.at[didx], qbuf, sem2)
        cp3 = pltpu.async_copy(v_hbm.at[sidx], vbuf, sem3)
        cp1.wait()
        cp2.wait()
        cp3.wait()

        perms = [jnp.bitwise_xor(lanes, sh) for sh in (8, 4, 2, 1)]

        def edge_body(i, ecarry):
            zvec = jnp.zeros((16,), jnp.float32)
            for h in range(H):
                kh = kbuf[i, pl.ds(h * 16, 16)]
                qh = qbuf[i, pl.ds(h * 16, 16)]
                sv = kh * qh
                for p in perms:
                    sv = sv + sv.at[p].get(mode="promise_in_bounds")
                sv = jnp.exp(jnp.clip(sv * 0.25, -5.0, 5.0))
                vh = vbuf[i, pl.ds(h * 16, 16)]
                obuf[i, pl.ds(h * 16, 16)] = vh * sv
                zvec = jnp.where(lanes == h, sv, zvec)
            obuf[i, pl.ds(D, 16)] = zvec
            return ecarry

        lax.fori_loop(0, CHUNK, edge_body, 0)
        pltpu.sync_copy(obuf, acc.at[didx], add=True)
        return carry

    lax.fori_loop(0, NCH, chunk_body, 0)
    plsc.subcore_barrier()
    for it in range(RBIT):
        b = it * NS + s
        @pl.when(b < NRB)
        def _():
            off = b * ZR
            pltpu.sync_copy(acc.at[pl.ds(off, ZR)],
                            out_hbm.at[c, pl.ds(off, ZR)])


def _edge_phase(src, dst, q, k, v):
    mesh = plsc.VectorSubcoreMesh(core_axis_name="c", subcore_axis_name="s")
    fn = functools.partial(
        pl.kernel,
        out_type=jax.ShapeDtypeStruct((NC, N, ROWW), jnp.float32),
        mesh=mesh,
        compiler_params=pltpu.CompilerParams(use_tc_tiling_on_sc=False),
        scratch_types=[
            pltpu.VMEM_SHARED((N, ROWW), jnp.float32),
        ],
    )(_edge_body)
    return fn(src, dst, q, k, v)


def _embed_body(h_ref, w_ref, o_ref):
    o_ref[...] = jnp.dot(h_ref[...], w_ref[...],
                         preferred_element_type=jnp.float32)


def _embed(h, w_emb_t):
    return pl.pallas_call(
        _embed_body,
        grid=(GRID,),
        in_specs=[
            pl.BlockSpec((BN, D), lambda i: (i, 0)),
            pl.BlockSpec((D, D), lambda i: (0, 0)),
        ],
        out_specs=pl.BlockSpec((BN, D), lambda i: (i, 0)),
        out_shape=jax.ShapeDtypeStruct((N, D), jnp.float32),
    )(h, w_emb_t)


def _qkv_body(x_ref, w_ref, b_ref, q_ref, k_ref, v_ref):
    y = jnp.dot(x_ref[...], w_ref[...],
                preferred_element_type=jnp.float32) + b_ref[...]
    q_ref[...] = y[:, :D]
    k_ref[...] = y[:, D:2 * D]
    v_ref[...] = y[:, 2 * D:]


def _qkv(x, w_qkv_t, b_qkv):
    out = jax.ShapeDtypeStruct((N, D), jnp.float32)
    return pl.pallas_call(
        _qkv_body,
        grid=(GRID,),
        in_specs=[
            pl.BlockSpec((BN, D), lambda i: (i, 0)),
            pl.BlockSpec((D, 3 * D), lambda i: (0, 0)),
            pl.BlockSpec((1, 3 * D), lambda i: (0, 0)),
        ],
        out_specs=[pl.BlockSpec((BN, D), lambda i: (i, 0))] * 3,
        out_shape=[out, out, out],
    )(x, w_qkv_t, b_qkv)


def _ln(x, g, b, eps=1e-5):
    m = jnp.mean(x, axis=-1, keepdims=True)
    v = jnp.mean(jnp.square(x - m), axis=-1, keepdims=True)
    return (x - m) * lax.rsqrt(v + eps) * g + b


def _node_body(x_ref, p_ref, wo_ref, bo_ref, g1_ref, bln1_ref,
               w1_ref, b1_ref, w2_ref, b2_ref, g2_ref, bln2_ref, o_ref):
    psum = p_ref[0] + p_ref[1]
    wv = psum[:, :D]
    jj = lax.broadcasted_iota(jnp.int32, (ROWW, D), 0)
    cc = lax.broadcasted_iota(jnp.int32, (ROWW, D), 1)
    sel = ((jj - D) == cc // DH).astype(jnp.float32)
    zb = jnp.dot(psum, sel, preferred_element_type=jnp.float32)
    attn = wv / (zb + 1e-6)
    o = jnp.dot(attn, wo_ref[...], preferred_element_type=jnp.float32)
    x2 = _ln(x_ref[...] + o + bo_ref[...], g1_ref[...], bln1_ref[...])
    y = jnp.dot(x2, w1_ref[...], preferred_element_type=jnp.float32)
    y = jnp.maximum(y + b1_ref[...], 0.0)
    y = jnp.dot(y, w2_ref[...], preferred_element_type=jnp.float32)
    o_ref[...] = _ln(x2 + y + b2_ref[...], g2_ref[...], bln2_ref[...])


def _node(x, parts, wo_t, bo, g1, bln1, w1_t, b1, w2_t, b2, g2, bln2):
    row = lambda i: (i, 0)
    full = lambda shape: pl.BlockSpec(shape, lambda i: (0, 0))
    return pl.pallas_call(
        _node_body,
        grid=(GRID,),
        in_specs=[
            pl.BlockSpec((BN, D), row),
            pl.BlockSpec((NC, BN, ROWW), lambda i: (0, i, 0)),
            full((D, D)), full((1, D)), full((1, D)), full((1, D)),
            full((D, 2 * D)), full((1, 2 * D)),
            full((2 * D, D)), full((1, D)), full((1, D)), full((1, D)),
        ],
        out_specs=pl.BlockSpec((BN, D), row),
        out_shape=jax.ShapeDtypeStruct((N, D), jnp.float32),
    )(x, parts, wo_t, bo, g1, bln1, w1_t, b1, w2_t, b2, g2, bln2)


def kernel(edge_index, h, W_emb, Wq, bq, Wk, bk, Wv, bv, Wo, bo,
           ln1_g, ln1_b, W1, b1, W2, b2, ln2_g, ln2_b):
    src = edge_index[0]
    dst = edge_index[1]
    x = _embed(h, W_emb.T)
    for l in range(2):
        w_qkv_t = jnp.concatenate([Wq[l].T, Wk[l].T, Wv[l].T], axis=1)
        b_qkv = jnp.concatenate([bq[l], bk[l], bv[l]])[None, :]
        q, k, v = _qkv(x, w_qkv_t, b_qkv)
        parts = _edge_phase(src, dst, q, k, v)
        x = _node(x, parts, Wo[l].T, bo[l][None, :],
                  ln1_g[l][None, :], ln1_b[l][None, :],
                  W1[l].T, b1[l][None, :], W2[l].T, b2[l][None, :],
                  ln2_g[l][None, :], ln2_b[l][None, :])
    return x


# speedup vs baseline: 85.4512x; 1.9039x over previous
import functools

import jax
import jax.numpy as jnp
from jax import lax
from jax.experimental import pallas as pl
from jax.experimental.pallas import tpu as pltpu
from jax.experimental.pallas import tpu_sc as plsc

N = 10000
E = 320000
D = 128
H = 8
DH = 16
ZPAD = 16
ROWW = D + ZPAD

NC = 2
NS = 16
NW = NC * NS
EPW = E // NW
CHUNK = 16
NCH = EPW // CHUNK
ZR = CHUNK
NRB = N // ZR
RBIT = -(-NRB // NS)
BN = 1000
GRID = N // BN


def _edge_body(src_hbm, dst_hbm, q_hbm, k_hbm, v_hbm, out_hbm, acc):
    pl.run_scoped(
        functools.partial(_edge_inner, src_hbm, dst_hbm, q_hbm,
                          k_hbm, v_hbm, out_hbm, acc),
        pltpu.VMEM((EPW,), jnp.int32),
        pltpu.VMEM((NCH, CHUNK), jnp.int32),
        pltpu.VMEM((CHUNK, D), jnp.float32),
        pltpu.VMEM((CHUNK, D), jnp.float32),
        pltpu.VMEM((CHUNK, D), jnp.float32),
        pltpu.VMEM((CHUNK, D), jnp.float32),
        pltpu.VMEM((CHUNK, D), jnp.float32),
        pltpu.VMEM((CHUNK, D), jnp.float32),
        pltpu.VMEM((CHUNK, ROWW), jnp.float32),
        pltpu.SemaphoreType.DMA,
        pltpu.SemaphoreType.DMA,
        pltpu.SemaphoreType.DMA,
    )


def _edge_inner(src_hbm, dst_hbm, q_hbm, k_hbm, v_hbm, out_hbm, acc,
                sidx, didx3, kb0, qb0, vb0, kb1, qb1, vb1, obuf,
                gsem0, gsem1, ssem):
    kqvs = ((kb0, qb0, vb0), (kb1, qb1, vb1))
    gsems = (gsem0, gsem1)
    c = lax.axis_index("c")
    s = lax.axis_index("s")
    w = c * NS + s
    lanes = lax.broadcasted_iota(jnp.int32, (16,), 0)
    lane_of = [8 * (h & 1) + 4 * ((h >> 1) & 1) + 2 * ((h >> 2) & 1)
               for h in range(H)]
    zero16 = lanes * 0
    bcast = [zero16 + lane_of[h] for h in range(H)]
    zperm = (8 * (lanes & 1) + 4 * ((lanes >> 1) & 1)
             + 2 * ((lanes >> 2) & 1))
    masks = {sh: (lanes & sh) == 0 for sh in (8, 4, 2)}
    perms = {sh: jnp.bitwise_xor(lanes, sh) for sh in (8, 4, 2, 1)}

    def _p(x, idx):
        return x.at[idx].get(mode="promise_in_bounds")

    def _merge(a, b, sh):
        t1 = jnp.where(masks[sh], a, b)
        t2 = jnp.where(masks[sh], b, a)
        return t1 + _p(t2, perms[sh])

    def _gissue(g, b):
        sl = pl.ds(g * CHUNK, CHUNK)
        kb, qb, vb = kqvs[b]
        pltpu.async_copy(k_hbm.at[sidx.at[sl]], kb, gsems[b])
        pltpu.async_copy(q_hbm.at[didx3.at[g]], qb, gsems[b])
        pltpu.async_copy(v_hbm.at[sidx.at[sl]], vb, gsems[b])

    def _gwait(b):
        kb, qb, vb = kqvs[b]
        sl0 = pl.ds(0, CHUNK)
        pltpu.make_async_copy(k_hbm.at[sidx.at[sl0]], kb, gsems[b]).wait()
        pltpu.make_async_copy(q_hbm.at[didx3.at[0]], qb, gsems[b]).wait()
        pltpu.make_async_copy(v_hbm.at[sidx.at[sl0]], vb, gsems[b]).wait()

    def _sissue(g):
        pltpu.async_copy(obuf, acc.at[didx3.at[g]], ssem, add=True)

    def _swait():
        pltpu.make_async_copy(obuf, acc.at[didx3.at[0]], ssem).wait()

    pltpu.sync_copy(src_hbm.at[w], sidx)
    pltpu.sync_copy(dst_hbm.at[w], didx3)

    def zrow(i, carry):
        for j in range(ROWW // 16):
            obuf[i, pl.ds(j * 16, 16)] = jnp.zeros((16,), jnp.float32)
        return carry

    lax.fori_loop(0, CHUNK, zrow, 0)
    for it in range(RBIT):
        rb = it * NS + s
        @pl.when(rb < NRB)
        def _():
            pltpu.sync_copy(obuf, acc.at[pl.ds(rb * ZR, ZR)])
    plsc.subcore_barrier()

    def compute(kqv, obuf):
        kb, qb, vb = kqv

        def edge_body(i, ecarry):
            p = [kb[i, pl.ds(h * 16, 16)] * qb[i, pl.ds(h * 16, 16)]
                 for h in range(H)]
            m1 = [_merge(p[2 * j], p[2 * j + 1], 8) for j in range(4)]
            m2 = [_merge(m1[0], m1[1], 4), _merge(m1[2], m1[3], 4)]
            e = _merge(m2[0], m2[1], 2)
            f = e + _p(e, perms[1])
            sc = jnp.exp(jnp.clip(f, -5.0, 5.0))
            obuf[i, pl.ds(D, 16)] = _p(sc, zperm)
            for h in range(H):
                vh = vb[i, pl.ds(h * 16, 16)]
                obuf[i, pl.ds(h * 16, 16)] = vh * _p(sc, bcast[h])
            return ecarry

        lax.fori_loop(0, CHUNK, edge_body, 0)

    _gissue(0, 0)

    def outer(g0, carry):
        for b in (0, 1):
            gg = 2 * g0 + b
            nb = 1 - b

            @pl.when(gg < NCH)
            def _():
                @pl.when(gg + 1 < NCH)
                def _():
                    _gissue(gg + 1, nb)

                _gwait(b)

                @pl.when(gg >= 1)
                def _():
                    _swait()

                compute(kqvs[b], obuf)
                _sissue(gg)
        return carry

    lax.fori_loop(0, (NCH + 1) // 2, outer, 0)
    _swait()
    plsc.subcore_barrier()
    for it in range(RBIT):
        rb = it * NS + s
        @pl.when(rb < NRB)
        def _():
            off = rb * ZR
            pltpu.sync_copy(acc.at[pl.ds(off, ZR)],
                            out_hbm.at[c, pl.ds(off, ZR)])


def _edge_phase(src, dst, q, k, v):
    mesh = plsc.VectorSubcoreMesh(core_axis_name="c", subcore_axis_name="s")
    fn = functools.partial(
        pl.kernel,
        out_type=jax.ShapeDtypeStruct((NC, N, ROWW), jnp.float32),
        mesh=mesh,
        compiler_params=pltpu.CompilerParams(use_tc_tiling_on_sc=False,
                                             internal_scratch_in_bytes=4096),
        scratch_types=[
            pltpu.VMEM_SHARED((N, ROWW), jnp.float32),
        ],
    )(_edge_body)
    return fn(src.reshape(NW, EPW), dst.reshape(NW, NCH, CHUNK), q, k, v)


def _embed_body(h_ref, w_ref, o_ref):
    o_ref[...] = jnp.dot(h_ref[...], w_ref[...],
                         preferred_element_type=jnp.float32)


def _embed(h, w_emb_t):
    return pl.pallas_call(
        _embed_body,
        grid=(GRID,),
        in_specs=[
            pl.BlockSpec((BN, D), lambda i: (i, 0)),
            pl.BlockSpec((D, D), lambda i: (0, 0)),
        ],
        out_specs=pl.BlockSpec((BN, D), lambda i: (i, 0)),
        out_shape=jax.ShapeDtypeStruct((N, D), jnp.float32),
    )(h, w_emb_t)


def _qkv_body(x_ref, w_ref, b_ref, q_ref, k_ref, v_ref):
    y = jnp.dot(x_ref[...], w_ref[...],
                preferred_element_type=jnp.float32) + b_ref[...]
    q_ref[...] = y[:, :D]
    k_ref[...] = y[:, D:2 * D] * 0.25
    v_ref[...] = y[:, 2 * D:]


def _qkv(x, w_qkv_t, b_qkv):
    out = jax.ShapeDtypeStruct((N, D), jnp.float32)
    return pl.pallas_call(
        _qkv_body,
        grid=(GRID,),
        in_specs=[
            pl.BlockSpec((BN, D), lambda i: (i, 0)),
            pl.BlockSpec((D, 3 * D), lambda i: (0, 0)),
            pl.BlockSpec((1, 3 * D), lambda i: (0, 0)),
        ],
        out_specs=[pl.BlockSpec((BN, D), lambda i: (i, 0))] * 3,
        out_shape=[out, out, out],
    )(x, w_qkv_t, b_qkv)


def _ln(x, g, b, eps=1e-5):
    m = jnp.mean(x, axis=-1, keepdims=True)
    v = jnp.mean(jnp.square(x - m), axis=-1, keepdims=True)
    return (x - m) * lax.rsqrt(v + eps) * g + b


def _node_body(x_ref, p_ref, wo_ref, bo_ref, g1_ref, bln1_ref,
               w1_ref, b1_ref, w2_ref, b2_ref, g2_ref, bln2_ref, o_ref):
    psum = p_ref[0] + p_ref[1]
    wv = psum[:, :D]
    jj = lax.broadcasted_iota(jnp.int32, (ROWW, D), 0)
    cc = lax.broadcasted_iota(jnp.int32, (ROWW, D), 1)
    sel = ((jj - D) == cc // DH).astype(jnp.float32)
    zb = jnp.dot(psum, sel, preferred_element_type=jnp.float32)
    attn = wv / (zb + 1e-6)
    o = jnp.dot(attn, wo_ref[...], preferred_element_type=jnp.float32)
    x2 = _ln(x_ref[...] + o + bo_ref[...], g1_ref[...], bln1_ref[...])
    y = jnp.dot(x2, w1_ref[...], preferred_element_type=jnp.float32)
    y = jnp.maximum(y + b1_ref[...], 0.0)
    y = jnp.dot(y, w2_ref[...], preferred_element_type=jnp.float32)
    o_ref[...] = _ln(x2 + y + b2_ref[...], g2_ref[...], bln2_ref[...])


def _node(x, parts, wo_t, bo, g1, bln1, w1_t, b1, w2_t, b2, g2, bln2):
    row = lambda i: (i, 0)
    full = lambda shape: pl.BlockSpec(shape, lambda i: (0, 0))
    return pl.pallas_call(
        _node_body,
        grid=(GRID,),
        in_specs=[
            pl.BlockSpec((BN, D), row),
            pl.BlockSpec((NC, BN, ROWW), lambda i: (0, i, 0)),
            full((D, D)), full((1, D)), full((1, D)), full((1, D)),
            full((D, 2 * D)), full((1, 2 * D)),
            full((2 * D, D)), full((1, D)), full((1, D)), full((1, D)),
        ],
        out_specs=pl.BlockSpec((BN, D), row),
        out_shape=jax.ShapeDtypeStruct((N, D), jnp.float32),
    )(x, parts, wo_t, bo, g1, bln1, w1_t, b1, w2_t, b2, g2, bln2)


def kernel(edge_index, h, W_emb, Wq, bq, Wk, bk, Wv, bv, Wo, bo,
           ln1_g, ln1_b, W1, b1, W2, b2, ln2_g, ln2_b):
    src = edge_index[0]
    dst = edge_index[1]
    x = _embed(h, W_emb.T)
    for l in range(2):
        w_qkv_t = jnp.concatenate([Wq[l].T, Wk[l].T, Wv[l].T], axis=1)
        b_qkv = jnp.concatenate([bq[l], bk[l], bv[l]])[None, :]
        q, k, v = _qkv(x, w_qkv_t, b_qkv)
        parts = _edge_phase(src, dst, q, k, v)
        x = _node(x, parts, Wo[l].T, bo[l][None, :],
                  ln1_g[l][None, :], ln1_b[l][None, :],
                  W1[l].T, b1[l][None, :], W2[l].T, b2[l][None, :],
                  ln2_g[l][None, :], ln2_b[l][None, :])
    return x


# speedup vs baseline: 106.2417x; 1.2433x over previous
import functools

import jax
import jax.numpy as jnp
from jax import lax
from jax.experimental import pallas as pl
from jax.experimental.pallas import tpu as pltpu
from jax.experimental.pallas import tpu_sc as plsc

N = 10000
E = 320000
D = 128
H = 8
DH = 16
ZPAD = 16
ROWW = D + ZPAD

NC = 2
NS = 16
NW = NC * NS
EPW = E // NW
CHUNK = 32
EPWP = 10016
NCH = EPWP // CHUNK
PADN = 10016
ZR = CHUNK
NRB = PADN // ZR
RBIT = -(-NRB // NS)
BN = 1000
GRID = N // BN


def _edge_body(src_hbm, dst_hbm, q_hbm, kv_hbm, out_hbm, acc):
    pl.run_scoped(
        functools.partial(_edge_inner, src_hbm, dst_hbm, q_hbm,
                          kv_hbm, out_hbm, acc),
        pltpu.VMEM((2, CHUNK), jnp.int32),
        pltpu.VMEM((NCH, CHUNK), jnp.int32),
        pltpu.VMEM((CHUNK, 2 * D), jnp.float32),
        pltpu.VMEM((CHUNK, D), jnp.float32),
        pltpu.VMEM((CHUNK, 2 * D), jnp.float32),
        pltpu.VMEM((CHUNK, D), jnp.float32),
        pltpu.VMEM((CHUNK, ROWW), jnp.float32),
        pltpu.SemaphoreType.DMA,
        pltpu.SemaphoreType.DMA,
        pltpu.SemaphoreType.DMA,
        pltpu.SemaphoreType.DMA,
        pltpu.SemaphoreType.DMA,
    )


def _edge_inner(src_hbm, dst_hbm, q_hbm, kv_hbm, out_hbm, acc,
                srcr, didx3, kvb0, qb0, kvb1, qb1, obuf,
                gsem0, gsem1, ssem, isem0, isem1):
    kqvs = ((kvb0, qb0), (kvb1, qb1))
    gsems = (gsem0, gsem1)
    isems = (isem0, isem1)
    c = lax.axis_index("c")
    s = lax.axis_index("s")
    w = c * NS + s
    lanes = lax.broadcasted_iota(jnp.int32, (16,), 0)
    lane_of = [8 * (h & 1) + 4 * ((h >> 1) & 1) + 2 * ((h >> 2) & 1)
               for h in range(H)]
    zero16 = lanes * 0
    bcast = [zero16 + lane_of[h] for h in range(H)]
    zperm = (8 * (lanes & 1) + 4 * ((lanes >> 1) & 1)
             + 2 * ((lanes >> 2) & 1))
    masks = {sh: (lanes & sh) == 0 for sh in (8, 4, 2)}
    perms = {sh: jnp.bitwise_xor(lanes, sh) for sh in (8, 4, 2, 1)}

    def _p(x, idx):
        return x.at[idx].get(mode="promise_in_bounds")

    def _merge(a, b, sh):
        t1 = jnp.where(masks[sh], a, b)
        t2 = jnp.where(masks[sh], b, a)
        return t1 + _p(t2, perms[sh])

    def _gissue(g, b):
        kvb, qb = kqvs[b]
        pltpu.async_copy(kv_hbm.at[srcr.at[b]], kvb, gsems[b])
        pltpu.async_copy(q_hbm.at[didx3.at[g]], qb, gsems[b])

    def _gwait(b):
        kvb, qb = kqvs[b]
        pltpu.make_async_copy(kv_hbm.at[srcr.at[b]], kvb, gsems[b]).wait()
        pltpu.make_async_copy(q_hbm.at[didx3.at[0]], qb, gsems[b]).wait()

    def _iissue(g, b):
        pltpu.async_copy(src_hbm.at[w, g], srcr.at[b], isems[b])

    def _iwait(b):
        pltpu.make_async_copy(src_hbm.at[w, 0], srcr.at[b],
                              isems[b]).wait()

    def _sissue(g):
        pltpu.async_copy(obuf, acc.at[didx3.at[g]], ssem, add=True)

    def _swait():
        pltpu.make_async_copy(obuf, acc.at[didx3.at[0]], ssem).wait()

    pltpu.sync_copy(dst_hbm.at[w], didx3)

    def zrow(i, carry):
        for j in range(ROWW // 16):
            obuf[i, pl.ds(j * 16, 16)] = jnp.zeros((16,), jnp.float32)
        return carry

    lax.fori_loop(0, CHUNK, zrow, 0)
    for it in range(RBIT):
        rb = it * NS + s
        @pl.when(rb < NRB)
        def _():
            pltpu.sync_copy(obuf, acc.at[pl.ds(rb * ZR, ZR)])
    plsc.subcore_barrier()

    def compute(kqv, obuf):
        kvb, qb = kqv

        def edge_body(i, ecarry):
            p = [kvb[i, pl.ds(h * 16, 16)] * qb[i, pl.ds(h * 16, 16)]
                 for h in range(H)]
            m1 = [_merge(p[2 * j], p[2 * j + 1], 8) for j in range(4)]
            m2 = [_merge(m1[0], m1[1], 4), _merge(m1[2], m1[3], 4)]
            e = _merge(m2[0], m2[1], 2)
            f = e + _p(e, perms[1])
            sc = jnp.exp(jnp.clip(f, -5.0, 5.0))
            obuf[i, pl.ds(D, 16)] = _p(sc, zperm)
            for h in range(H):
                vh = kvb[i, pl.ds(D + h * 16, 16)]
                obuf[i, pl.ds(h * 16, 16)] = vh * _p(sc, bcast[h])
            return ecarry

        lax.fori_loop(0, CHUNK, edge_body, 0)

    pltpu.sync_copy(src_hbm.at[w, 0], srcr.at[0])
    _iissue(1, 1)
    _gissue(0, 0)

    def outer(g0, carry):
        for b in (0, 1):
            gg = 2 * g0 + b
            nb = 1 - b

            @pl.when(gg < NCH)
            def _():
                @pl.when(gg + 1 < NCH)
                def _():
                    _iwait(nb)
                    _gissue(gg + 1, nb)

                _gwait(b)

                @pl.when(gg + 2 < NCH)
                def _():
                    _iissue(gg + 2, b)

                @pl.when(gg >= 1)
                def _():
                    _swait()

                compute(kqvs[b], obuf)
                _sissue(gg)
        return carry

    lax.fori_loop(0, (NCH + 1) // 2, outer, 0)
    _swait()
    plsc.subcore_barrier()
    for it in range(RBIT):
        rb = it * NS + s
        @pl.when(rb < NRB)
        def _():
            off = rb * ZR
            pltpu.sync_copy(acc.at[pl.ds(off, ZR)],
                            out_hbm.at[c, pl.ds(off, ZR)])


def _edge_phase(src, dst, q, kv):
    mesh = plsc.VectorSubcoreMesh(core_axis_name="c", subcore_axis_name="s")
    fn = functools.partial(
        pl.kernel,
        out_type=jax.ShapeDtypeStruct((NC, PADN, ROWW), jnp.float32),
        mesh=mesh,
        compiler_params=pltpu.CompilerParams(use_tc_tiling_on_sc=False,
                                             internal_scratch_in_bytes=4096),
        scratch_types=[
            pltpu.VMEM_SHARED((PADN, ROWW), jnp.float32),
        ],
    )(_edge_body)
    pad = EPWP - EPW
    srcp = jnp.concatenate(
        [src.reshape(NW, EPW),
         jnp.zeros((NW, pad), jnp.int32)], axis=1).reshape(NW, NCH, CHUNK)
    dstp = jnp.concatenate(
        [dst.reshape(NW, EPW),
         jnp.full((NW, pad), PADN - 1, jnp.int32)], axis=1
    ).reshape(NW, NCH, CHUNK)
    return fn(srcp, dstp, q, kv)


def _embed_body(h_ref, w_ref, o_ref):
    o_ref[...] = jnp.dot(h_ref[...], w_ref[...],
                         preferred_element_type=jnp.float32)


def _embed(h, w_emb_t):
    return pl.pallas_call(
        _embed_body,
        grid=(GRID,),
        in_specs=[
            pl.BlockSpec((BN, D), lambda i: (i, 0)),
            pl.BlockSpec((D, D), lambda i: (0, 0)),
        ],
        out_specs=pl.BlockSpec((BN, D), lambda i: (i, 0)),
        out_shape=jax.ShapeDtypeStruct((N, D), jnp.float32),
    )(h, w_emb_t)


def _qkv_body(x_ref, w_ref, b_ref, q_ref, kv_ref):
    y = jnp.dot(x_ref[...], w_ref[...],
                preferred_element_type=jnp.float32) + b_ref[...]
    q_ref[...] = y[:, :D]
    kv_ref[:, :D] = y[:, D:2 * D] * 0.25
    kv_ref[:, D:] = y[:, 2 * D:]


def _qkv(x, w_qkv_t, b_qkv):
    return pl.pallas_call(
        _qkv_body,
        grid=(GRID,),
        in_specs=[
            pl.BlockSpec((BN, D), lambda i: (i, 0)),
            pl.BlockSpec((D, 3 * D), lambda i: (0, 0)),
            pl.BlockSpec((1, 3 * D), lambda i: (0, 0)),
        ],
        out_specs=[pl.BlockSpec((BN, D), lambda i: (i, 0)),
                   pl.BlockSpec((BN, 2 * D), lambda i: (i, 0))],
        out_shape=[jax.ShapeDtypeStruct((N, D), jnp.float32),
                   jax.ShapeDtypeStruct((N, 2 * D), jnp.float32)],
    )(x, w_qkv_t, b_qkv)


def _ln(x, g, b, eps=1e-5):
    m = jnp.mean(x, axis=-1, keepdims=True)
    v = jnp.mean(jnp.square(x - m), axis=-1, keepdims=True)
    return (x - m) * lax.rsqrt(v + eps) * g + b


def _node_body(x_ref, p_ref, wo_ref, bo_ref, g1_ref, bln1_ref,
               w1_ref, b1_ref, w2_ref, b2_ref, g2_ref, bln2_ref, o_ref):
    psum = p_ref[0] + p_ref[1]
    wv = psum[:, :D]
    jj = lax.broadcasted_iota(jnp.int32, (ROWW, D), 0)
    cc = lax.broadcasted_iota(jnp.int32, (ROWW, D), 1)
    sel = ((jj - D) == cc // DH).astype(jnp.float32)
    zb = jnp.dot(psum, sel, preferred_element_type=jnp.float32)
    attn = wv / (zb + 1e-6)
    o = jnp.dot(attn, wo_ref[...], preferred_element_type=jnp.float32)
    x2 = _ln(x_ref[...] + o + bo_ref[...], g1_ref[...], bln1_ref[...])
    y = jnp.dot(x2, w1_ref[...], preferred_element_type=jnp.float32)
    y = jnp.maximum(y + b1_ref[...], 0.0)
    y = jnp.dot(y, w2_ref[...], preferred_element_type=jnp.float32)
    o_ref[...] = _ln(x2 + y + b2_ref[...], g2_ref[...], bln2_ref[...])


def _node(x, parts, wo_t, bo, g1, bln1, w1_t, b1, w2_t, b2, g2, bln2):
    row = lambda i: (i, 0)
    full = lambda shape: pl.BlockSpec(shape, lambda i: (0, 0))
    return pl.pallas_call(
        _node_body,
        grid=(GRID,),
        in_specs=[
            pl.BlockSpec((BN, D), row),
            pl.BlockSpec((NC, BN, ROWW), lambda i: (0, i, 0)),
            full((D, D)), full((1, D)), full((1, D)), full((1, D)),
            full((D, 2 * D)), full((1, 2 * D)),
            full((2 * D, D)), full((1, D)), full((1, D)), full((1, D)),
        ],
        out_specs=pl.BlockSpec((BN, D), row),
        out_shape=jax.ShapeDtypeStruct((N, D), jnp.float32),
    )(x, parts, wo_t, bo, g1, bln1, w1_t, b1, w2_t, b2, g2, bln2)


def kernel(edge_index, h, W_emb, Wq, bq, Wk, bk, Wv, bv, Wo, bo,
           ln1_g, ln1_b, W1, b1, W2, b2, ln2_g, ln2_b):
    src = edge_index[0]
    dst = edge_index[1]
    x = _embed(h, W_emb.T)
    for l in range(2):
        w_qkv_t = jnp.concatenate([Wq[l].T, Wk[l].T, Wv[l].T], axis=1)
        b_qkv = jnp.concatenate([bq[l], bk[l], bv[l]])[None, :]
        q, kv = _qkv(x, w_qkv_t, b_qkv)
        parts = _edge_phase(src, dst, q, kv)
        x = _node(x, parts, Wo[l].T, bo[l][None, :],
                  ln1_g[l][None, :], ln1_b[l][None, :],
                  W1[l].T, b1[l][None, :], W2[l].T, b2[l][None, :],
                  ln2_g[l][None, :], ln2_b[l][None, :])
    return x
